# baseline (speedup 1.0000x reference)
import functools

import jax
import jax.numpy as jnp
from jax import lax
from jax.experimental import pallas as pl
from jax.experimental.pallas import tpu as pltpu
from jax.experimental.pallas import tpu_sc as plsc

BS = 4096
NUM_AXIOMS = 200
N = BS * NUM_AXIOMS
EMBED_DIM = 64

NC = 2
NS = 16
NW = NC * NS
PW = N // NW
CHUNK = 512
NSTREAM = CHUNK // 128
NCHUNK = PW // CHUNK


def _body(emb_hbm, cl_hbm, cr_hbm, out_hbm,
          idxl_v, idxr_v, rowsl_v, rowsr_v, out_v, sem):
    wid = lax.axis_index("c") * NS + lax.axis_index("s")
    row0 = wid * (PW // 128)

    lanes = lax.iota(jnp.int32, 16)

    def chunk_body(c, _):
        r = row0 + c * NSTREAM
        pltpu.sync_copy(cl_hbm.at[pl.ds(r, NSTREAM)], idxl_v)
        pltpu.sync_copy(cr_hbm.at[pl.ds(r, NSTREAM)], idxr_v)

        copies = []
        for j in range(NSTREAM):
            copies.append(pltpu.async_copy(
                emb_hbm.at[idxl_v.at[j]],
                rowsl_v.at[pl.ds(j * 128, 128)], sem))
            copies.append(pltpu.async_copy(
                emb_hbm.at[idxr_v.at[j]],
                rowsr_v.at[pl.ds(j * 128, 128)], sem))
        for cp in copies:
            cp.wait()

        out_base = c * CHUNK

        def group_body(g, _):
            pair_ids = g * 16 + lanes
            accs = [jnp.zeros((16,), jnp.float32) for _ in range(4)]
            for d in range(EMBED_DIM):
                dcol = jnp.full((16,), d, jnp.int32)
                lv = plsc.load_gather(rowsl_v, [pair_ids, dcol])
                rv = plsc.load_gather(rowsr_v, [pair_ids, dcol])
                accs[d % 4] = accs[d % 4] + lv * rv
            acc = (accs[0] + accs[1]) + (accs[2] + accs[3])
            out_v[pl.ds(out_base + g * 16, 16)] = acc
            return _

        lax.fori_loop(0, CHUNK // 16, group_body, 0, unroll=False)
        return _

    lax.fori_loop(0, NCHUNK, chunk_body, 0, unroll=False)

    pltpu.sync_copy(out_v, out_hbm.at[pl.ds(wid * PW, PW)])


@jax.jit
def _scores(cl2, cr2, emb):
    mesh = plsc.VectorSubcoreMesh(
        core_axis_name="c", subcore_axis_name="s",
        num_cores=NC, num_subcores=NS)
    f = pl.kernel(
        _body,
        out_type=jax.ShapeDtypeStruct((N,), jnp.float32),
        mesh=mesh,
        scratch_types=[
            pltpu.VMEM((NSTREAM, 128), jnp.int32),
            pltpu.VMEM((NSTREAM, 128), jnp.int32),
            pltpu.VMEM((CHUNK, EMBED_DIM), jnp.float32),
            pltpu.VMEM((CHUNK, EMBED_DIM), jnp.float32),
            pltpu.VMEM((PW,), jnp.float32),
            pltpu.SemaphoreType.DMA,
        ],
        compiler_params=pltpu.CompilerParams(
            needs_layout_passes=False, use_tc_tiling_on_sc=False),
    )
    return f(emb, cl2, cr2)


def kernel(x, emb):
    bs, num_axioms, ents = x.shape
    xf = x.reshape(-1, ents).astype(jnp.int32)
    cl2 = xf[:, 0].reshape(N // 128, 128)
    cr2 = xf[:, 1].reshape(N // 128, 128)
    scores = _scores(cl2, cr2, emb)
    return scores.reshape(bs, num_axioms)


# speedup vs baseline: 1.9126x; 1.9126x over previous
import functools

import jax
import jax.numpy as jnp
from jax import lax
from jax.experimental import pallas as pl
from jax.experimental.pallas import tpu as pltpu
from jax.experimental.pallas import tpu_sc as plsc

BS = 4096
NUM_AXIOMS = 200
N = BS * NUM_AXIOMS
EMBED_DIM = 64

NC = 2
NS = 16
NW = NC * NS
PW = N // NW
CHUNK = 512
NSTREAM = CHUNK // 128
NCHUNK = PW // CHUNK


def _body(emb_hbm, cl_hbm, cr_hbm, out_hbm,
          idxl_v, idxr_v, rowsl_v, rowsr_v, out_v, sem):
    wid = lax.axis_index("c") * NS + lax.axis_index("s")
    row0 = wid * (PW // 128)

    lanes = lax.iota(jnp.int32, 16)
    lane15 = lanes == 15

    def chunk_body(c, _):
        r = row0 + c * NSTREAM
        pltpu.sync_copy(cl_hbm.at[pl.ds(r, NSTREAM)], idxl_v)
        pltpu.sync_copy(cr_hbm.at[pl.ds(r, NSTREAM)], idxr_v)

        copies = []
        for j in range(NSTREAM):
            copies.append(pltpu.async_copy(
                emb_hbm.at[idxl_v.at[j]],
                rowsl_v.at[pl.ds(j * 128, 128)], sem))
            copies.append(pltpu.async_copy(
                emb_hbm.at[idxr_v.at[j]],
                rowsr_v.at[pl.ds(j * 128, 128)], sem))
        for cp in copies:
            cp.wait()

        out_base = c * CHUNK

        def pair_body(p, _):
            prods = []
            for q in range(4):
                lv = rowsl_v[p, pl.ds(q * 16, 16)]
                rv = rowsr_v[p, pl.ds(q * 16, 16)]
                prods.append(lv * rv)
            part = (prods[0] + prods[1]) + (prods[2] + prods[3])
            cum = plsc.cumsum(part)
            idx = jnp.full((16,), out_base + p, jnp.int32)
            plsc.store_scatter(out_v, [idx], cum, mask=lane15)
            return _

        lax.fori_loop(0, CHUNK, pair_body, 0, unroll=4)
        return _

    lax.fori_loop(0, NCHUNK, chunk_body, 0, unroll=False)

    pltpu.sync_copy(out_v, out_hbm.at[pl.ds(wid * PW, PW)])


@jax.jit
def _scores(cl2, cr2, emb):
    mesh = plsc.VectorSubcoreMesh(
        core_axis_name="c", subcore_axis_name="s",
        num_cores=NC, num_subcores=NS)
    f = pl.kernel(
        _body,
        out_type=jax.ShapeDtypeStruct((N,), jnp.float32),
        mesh=mesh,
        scratch_types=[
            pltpu.VMEM((NSTREAM, 128), jnp.int32),
            pltpu.VMEM((NSTREAM, 128), jnp.int32),
            pltpu.VMEM((CHUNK, EMBED_DIM), jnp.float32),
            pltpu.VMEM((CHUNK, EMBED_DIM), jnp.float32),
            pltpu.VMEM((PW,), jnp.float32),
            pltpu.SemaphoreType.DMA,
        ],
        compiler_params=pltpu.CompilerParams(
            needs_layout_passes=False, use_tc_tiling_on_sc=False),
    )
    return f(emb, cl2, cr2)


def kernel(x, emb):
    bs, num_axioms, ents = x.shape
    xf = x.reshape(-1, ents).astype(jnp.int32)
    cl2 = xf[:, 0].reshape(N // 128, 128)
    cr2 = xf[:, 1].reshape(N // 128, 128)
    scores = _scores(cl2, cr2, emb)
    return scores.reshape(bs, num_axioms)
